# baseline (speedup 1.0000x reference)
import functools

import jax
import jax.numpy as jnp
from jax import lax
from jax.experimental import pallas as pl
from jax.experimental.pallas import tpu as pltpu
from jax.experimental.pallas import tpu_sc as plsc

N_UEMB = 1000000
N_CEMB = 100000
EMB_DIM = 64
BATCH = 16384

NUM_CORES = 2
NUM_SUBCORES = 16
LANES = 16
NW = NUM_CORES * NUM_SUBCORES
B_PER_W = BATCH // NW
CHUNK = 128
NCHUNK = B_PER_W // CHUNK
GROUPS_PER_CHUNK = CHUNK // LANES

_mesh = plsc.VectorSubcoreMesh(
    core_axis_name="c", subcore_axis_name="s",
    num_cores=NUM_CORES, num_subcores=NUM_SUBCORES,
)


@functools.partial(
    pl.kernel,
    out_type=jax.ShapeDtypeStruct((BATCH,), jnp.float32),
    mesh=_mesh,
    scratch_types=[
        pltpu.VMEM((NCHUNK, CHUNK), jnp.int32),
        pltpu.VMEM((NCHUNK, CHUNK), jnp.int32),
        pltpu.VMEM((NCHUNK, CHUNK, EMB_DIM), jnp.float32),
        pltpu.VMEM((NCHUNK, CHUNK, EMB_DIM), jnp.float32),
        pltpu.VMEM((NCHUNK, CHUNK), jnp.float32),
        pltpu.VMEM((NCHUNK, CHUNK), jnp.float32),
        pltpu.VMEM((B_PER_W,), jnp.float32),
        pltpu.VMEM((LANES, LANES + 1), jnp.float32),
        pltpu.SemaphoreType.DMA,
    ],
    compiler_params=pltpu.CompilerParams(
        needs_layout_passes=False, use_tc_tiling_on_sc=False),
)
def _ranker_sc(x1_hbm, x2_hbm, uemb_hbm, cemb_hbm, ubias_hbm, cbias_hbm,
               out_hbm, idx1_v, idx2_v, u_v, c_v, ub_v, cb_v, out_v, pad_v, sem):
    wid = lax.axis_index("s") * NUM_CORES + lax.axis_index("c")
    base = wid * B_PER_W

    for j in range(NCHUNK):
        pltpu.sync_copy(x1_hbm.at[pl.ds(base + j * CHUNK, CHUNK)], idx1_v.at[j])
        pltpu.sync_copy(x2_hbm.at[pl.ds(base + j * CHUNK, CHUNK)], idx2_v.at[j])

    copies = []
    for j in range(NCHUNK):
        copies.append(pltpu.async_copy(uemb_hbm.at[idx1_v.at[j]], u_v.at[j], sem))
        copies.append(pltpu.async_copy(cemb_hbm.at[idx2_v.at[j]], c_v.at[j], sem))
        copies.append(pltpu.async_copy(ubias_hbm.at[idx1_v.at[j]], ub_v.at[j], sem))
        copies.append(pltpu.async_copy(cbias_hbm.at[idx2_v.at[j]], cb_v.at[j], sem))
    for cp in copies:
        cp.wait()

    iota16 = lax.iota(jnp.int32, LANES)

    for j in range(NCHUNK):

        @pl.loop(0, GROUPS_PER_CHUNK)
        def _(g):
            rbase = g * LANES
            for i in range(LANES):
                r = rbase + i
                acc = u_v[j, r, pl.ds(0, LANES)] * c_v[j, r, pl.ds(0, LANES)]
                for k in range(1, EMB_DIM // LANES):
                    acc = acc + (u_v[j, r, pl.ds(k * LANES, LANES)]
                                 * c_v[j, r, pl.ds(k * LANES, LANES)])
                plsc.store_scatter(pad_v, [iota16, jnp.full((LANES,), i, jnp.int32)], acc)
            dots = pad_v[0, pl.ds(0, LANES)]
            for l in range(1, LANES):
                dots = dots + pad_v[l, pl.ds(0, LANES)]
            ub = ub_v[j, pl.ds(rbase, LANES)]
            cb = cb_v[j, pl.ds(rbase, LANES)]
            tot = dots + float(EMB_DIM) * (ub + cb)
            out_v[pl.ds(j * CHUNK + rbase, LANES)] = 1.0 / (1.0 + jnp.exp(-tot))

    pltpu.sync_copy(out_v, out_hbm.at[pl.ds(base, B_PER_W)])


def kernel(x1, x2, uemb, cemb, user_bias, creator_bias):
    x1 = x1.astype(jnp.int32)
    x2 = x2.astype(jnp.int32)
    return _ranker_sc(x1, x2, uemb, cemb,
                      user_bias.reshape(-1), creator_bias.reshape(-1))
